# Initial kernel scaffold; baseline (speedup 1.0000x reference)
import jax
import jax.numpy as jnp
from jax.experimental import pallas as pl


def kernel(coords, rbf, node_is, node_js, emb_table, W_rbf0, b_rbf0, W_lin, b_lin, W_rbf1):
    raise NotImplementedError("write your pallas kernel here")


# speedup vs baseline: 1.1272x; 1.1272x over previous
import functools

import jax
import jax.numpy as jnp
from jax import lax
from jax.experimental import pallas as pl
from jax.experimental.pallas import tpu as pltpu
from jax.experimental.pallas import tpu_sc as plsc

E = 800_000
N_NODES = 50_000
H = 64
NR = 16
NT = 95

NW = 32
W_CNT = 25_088
EP = NW * W_CNT
SC_UNROLL = 8

ET = 4_000


def _sc_body(coords_hbm, nis_hbm, njs_hbm, ci_hbm, cj_hbm, tab_v, idx_v, out_v):
    wid = lax.axis_index("s") * 2 + lax.axis_index("c")
    base = wid * W_CNT
    pltpu.sync_copy(coords_hbm, tab_v)
    for src, dst in ((nis_hbm, ci_hbm), (njs_hbm, cj_hbm)):
        pltpu.sync_copy(src.at[pl.ds(base, W_CNT)], idx_v)

        def body(t, _):
            for u in range(SC_UNROLL):
                off = (t * SC_UNROLL + u) * 16
                idx16 = idx_v[pl.ds(off, 16)]
                out_v[pl.ds(off, 16)] = plsc.load_gather(tab_v, [idx16])
            return 0

        lax.fori_loop(0, W_CNT // (16 * SC_UNROLL), body, 0)
        pltpu.sync_copy(out_v, dst.at[pl.ds(base, W_CNT)])


def _sc_gather(coords, nis_p, njs_p):
    call = functools.partial(
        pl.kernel,
        mesh=plsc.VectorSubcoreMesh(core_axis_name="c", subcore_axis_name="s"),
        compiler_params=pltpu.CompilerParams(needs_layout_passes=False),
        out_type=(
            jax.ShapeDtypeStruct((EP,), jnp.int32),
            jax.ShapeDtypeStruct((EP,), jnp.int32),
        ),
        scratch_types=[
            pltpu.VMEM((N_NODES,), jnp.int32),
            pltpu.VMEM((W_CNT,), jnp.int32),
            pltpu.VMEM((W_CNT,), jnp.int32),
        ],
    )(_sc_body)
    return call(coords, nis_p, njs_p)


def _prep_body(embp_ref, wlin_ref, pcat_ref):
    embp = embp_ref[...]
    pcat_ref[0:128, :] = jnp.dot(
        embp, wlin_ref[0:64, :], preferred_element_type=jnp.float32, precision=lax.Precision.HIGHEST
    )
    pcat_ref[128:256, :] = jnp.dot(
        embp, wlin_ref[64:128, :], preferred_element_type=jnp.float32, precision=lax.Precision.HIGHEST
    )


def _edge_body(ci_ref, cj_ref, rbf_ref, pcat_ref, wc_ref, b0_ref, wlin_ref,
               bl_ref, e1_ref, e2_ref):
    col = lax.broadcasted_iota(jnp.int16, (ET, 256), 1)
    cif = ci_ref[...].astype(jnp.int16)
    cjf = cj_ref[...].astype(jnp.int16) + jnp.int16(128)
    sel = (col == cif) | (col == cjf)
    oh = jnp.where(sel, jnp.bfloat16(1), jnp.bfloat16(0))
    pc = pcat_ref[...]
    pc_hi = pc.astype(jnp.bfloat16)
    pc_lo = (pc - pc_hi.astype(jnp.float32)).astype(jnp.bfloat16)
    g = (jnp.dot(oh, pc_hi, preferred_element_type=jnp.float32)
         + jnp.dot(oh, pc_lo, preferred_element_type=jnp.float32))

    rbf = rbf_ref[...]
    h = jnp.dot(rbf, wc_ref[...], preferred_element_type=jnp.float32, precision=lax.Precision.HIGHEST)
    h0 = h[:, 0:64] + b0_ref[...]
    rbf0 = h0 * (1.0 / (1.0 + jnp.exp(-h0)))

    h1 = g + jnp.dot(rbf0, wlin_ref[128:192, :],
                     preferred_element_type=jnp.float32, precision=lax.Precision.HIGHEST) + bl_ref[...]
    e1 = h1 * (1.0 / (1.0 + jnp.exp(-h1)))
    e1_ref[...] = e1
    e2_ref[...] = h[:, 64:128] * e1


def _edge_call(ci, cj, rbf, pcat, wc, b0, wlin, bl):
    grid = (E // ET,)
    return pl.pallas_call(
        _edge_body,
        grid=grid,
        in_specs=[
            pl.BlockSpec((ET, 1), lambda i: (i, 0)),
            pl.BlockSpec((ET, 1), lambda i: (i, 0)),
            pl.BlockSpec((ET, NR), lambda i: (i, 0)),
            pl.BlockSpec((256, H), lambda i: (0, 0)),
            pl.BlockSpec((NR, 128), lambda i: (0, 0)),
            pl.BlockSpec((1, H), lambda i: (0, 0)),
            pl.BlockSpec((3 * H, H), lambda i: (0, 0)),
            pl.BlockSpec((1, H), lambda i: (0, 0)),
        ],
        out_specs=[
            pl.BlockSpec((ET, H), lambda i: (i, 0)),
            pl.BlockSpec((ET, H), lambda i: (i, 0)),
        ],
        out_shape=[
            jax.ShapeDtypeStruct((E, H), jnp.float32),
            jax.ShapeDtypeStruct((E, H), jnp.float32),
        ],
    )(ci, cj, rbf, pcat, wc, b0, wlin, bl)


def kernel(coords, rbf, node_is, node_js, emb_table, W_rbf0, b_rbf0, W_lin,
           b_lin, W_rbf1):
    nis_p = jnp.concatenate(
        [node_is, jnp.zeros((EP - E,), node_is.dtype)]).astype(jnp.int32)
    njs_p = jnp.concatenate(
        [node_js, jnp.zeros((EP - E,), node_js.dtype)]).astype(jnp.int32)
    ci_p, cj_p = _sc_gather(coords.astype(jnp.int32), nis_p, njs_p)
    ci = ci_p[:E].reshape(E, 1)
    cj = cj_p[:E].reshape(E, 1)

    embp = jnp.concatenate(
        [emb_table, jnp.zeros((128 - NT, H), emb_table.dtype)])
    pcat = pl.pallas_call(
        _prep_body,
        out_shape=jax.ShapeDtypeStruct((256, H), jnp.float32),
    )(embp, W_lin)

    wc = jnp.concatenate([W_rbf0, W_rbf1], axis=1)
    b0 = b_rbf0.reshape(1, H)
    bl = b_lin.reshape(1, H)
    return _edge_call(ci, cj, rbf, pcat, wc, b0, W_lin, bl)


# speedup vs baseline: 1.6983x; 1.5067x over previous
import functools

import jax
import jax.numpy as jnp
from jax import lax
from jax.experimental import pallas as pl
from jax.experimental.pallas import tpu as pltpu
from jax.experimental.pallas import tpu_sc as plsc

E = 800_000
N_NODES = 50_000
H = 64
NR = 16
NT = 95

NW = 32
W_CNT = 25_088
EP = NW * W_CNT
SC_UNROLL = 8

ET = 8_000


def _sc_body(coords_hbm, nis_hbm, njs_hbm, ci_hbm, cj_hbm, tab_v, idx_v, out_v):
    wid = lax.axis_index("s") * 2 + lax.axis_index("c")
    base = wid * W_CNT
    pltpu.sync_copy(coords_hbm, tab_v)
    for src, dst in ((nis_hbm, ci_hbm), (njs_hbm, cj_hbm)):
        pltpu.sync_copy(src.at[pl.ds(base, W_CNT)], idx_v)

        def body(t, _):
            for u in range(SC_UNROLL):
                off = (t * SC_UNROLL + u) * 16
                idx16 = idx_v[pl.ds(off, 16)]
                out_v[pl.ds(off, 16)] = plsc.load_gather(tab_v, [idx16])
            return 0

        lax.fori_loop(0, W_CNT // (16 * SC_UNROLL), body, 0)
        pltpu.sync_copy(out_v, dst.at[pl.ds(base, W_CNT)])


def _sc_gather(coords, nis_p, njs_p):
    call = functools.partial(
        pl.kernel,
        mesh=plsc.VectorSubcoreMesh(core_axis_name="c", subcore_axis_name="s"),
        compiler_params=pltpu.CompilerParams(needs_layout_passes=False),
        out_type=(
            jax.ShapeDtypeStruct((EP,), jnp.int32),
            jax.ShapeDtypeStruct((EP,), jnp.int32),
        ),
        scratch_types=[
            pltpu.VMEM((N_NODES,), jnp.int32),
            pltpu.VMEM((W_CNT,), jnp.int32),
            pltpu.VMEM((W_CNT,), jnp.int32),
        ],
    )(_sc_body)
    return call(coords, nis_p, njs_p)


def _prep_body(embp_ref, wlin_ref, pcat_ref):
    embp = embp_ref[...]
    pcat_ref[0:128, :] = jnp.dot(
        embp, wlin_ref[0:64, :], preferred_element_type=jnp.float32, precision=lax.Precision.HIGHEST
    )
    pcat_ref[128:256, :] = jnp.dot(
        embp, wlin_ref[64:128, :], preferred_element_type=jnp.float32, precision=lax.Precision.HIGHEST
    )


def _edge_body(ci_ref, cj_ref, rbf_ref, pcat_ref, wc_ref, b0_ref, wlin_ref,
               bl_ref, e1_ref, e2_ref):
    col = lax.broadcasted_iota(jnp.int16, (ET, 256), 1)
    cif = ci_ref[...].astype(jnp.int16)
    cjf = cj_ref[...].astype(jnp.int16) + jnp.int16(128)
    sel = (col == cif) | (col == cjf)
    oh = jnp.where(sel, jnp.bfloat16(1), jnp.bfloat16(0))
    pc_hi = pcat_ref[...].astype(jnp.bfloat16)
    g = jnp.dot(oh, pc_hi, preferred_element_type=jnp.float32)

    rbf = rbf_ref[...]
    h = jnp.dot(rbf, wc_ref[...], preferred_element_type=jnp.float32)
    h0 = h[:, 0:64] + b0_ref[...]
    rbf0 = h0 * (1.0 / (1.0 + jnp.exp(-h0)))

    h1 = g + jnp.dot(rbf0, wlin_ref[128:192, :],
                     preferred_element_type=jnp.float32) + bl_ref[...]
    e1 = h1 * (1.0 / (1.0 + jnp.exp(-h1)))
    e1_ref[...] = e1
    e2_ref[...] = h[:, 64:128] * e1


def _edge_call(ci, cj, rbf, pcat, wc, b0, wlin, bl):
    grid = (E // ET,)
    return pl.pallas_call(
        _edge_body,
        grid=grid,
        in_specs=[
            pl.BlockSpec((ET, 1), lambda i: (i, 0)),
            pl.BlockSpec((ET, 1), lambda i: (i, 0)),
            pl.BlockSpec((ET, NR), lambda i: (i, 0)),
            pl.BlockSpec((256, H), lambda i: (0, 0)),
            pl.BlockSpec((NR, 128), lambda i: (0, 0)),
            pl.BlockSpec((1, H), lambda i: (0, 0)),
            pl.BlockSpec((3 * H, H), lambda i: (0, 0)),
            pl.BlockSpec((1, H), lambda i: (0, 0)),
        ],
        out_specs=[
            pl.BlockSpec((ET, H), lambda i: (i, 0)),
            pl.BlockSpec((ET, H), lambda i: (i, 0)),
        ],
        out_shape=[
            jax.ShapeDtypeStruct((E, H), jnp.float32),
            jax.ShapeDtypeStruct((E, H), jnp.float32),
        ],
    )(ci, cj, rbf, pcat, wc, b0, wlin, bl)


def kernel(coords, rbf, node_is, node_js, emb_table, W_rbf0, b_rbf0, W_lin,
           b_lin, W_rbf1):
    nis_p = jnp.concatenate(
        [node_is, jnp.zeros((EP - E,), node_is.dtype)]).astype(jnp.int32)
    njs_p = jnp.concatenate(
        [node_js, jnp.zeros((EP - E,), node_js.dtype)]).astype(jnp.int32)
    ci_p, cj_p = _sc_gather(coords.astype(jnp.int32), nis_p, njs_p)
    ci = ci_p[:E].reshape(E, 1)
    cj = cj_p[:E].reshape(E, 1)

    embp = jnp.concatenate(
        [emb_table, jnp.zeros((128 - NT, H), emb_table.dtype)])
    pcat = pl.pallas_call(
        _prep_body,
        out_shape=jax.ShapeDtypeStruct((256, H), jnp.float32),
    )(embp, W_lin)

    wc = jnp.concatenate([W_rbf0, W_rbf1], axis=1)
    b0 = b_rbf0.reshape(1, H)
    bl = b_lin.reshape(1, H)
    return _edge_call(ci, cj, rbf, pcat, wc, b0, W_lin, bl)


# speedup vs baseline: 2.8581x; 1.6829x over previous
import functools

import jax
import jax.numpy as jnp
from jax import lax
from jax.experimental import pallas as pl
from jax.experimental.pallas import tpu as pltpu
from jax.experimental.pallas import tpu_sc as plsc

E = 800_000
N_NODES = 50_000
H = 64
NR = 16
NT = 95

NW = 32
W_CNT = 25_088
EP = NW * W_CNT
SC_UNROLL = 8

ET = 6_400
KROWS = ET // 128


def _sc_body(coords_hbm, nis_hbm, njs_hbm, key_hbm, tab_v, idx_v, out_v):
    wid = lax.axis_index("s") * 2 + lax.axis_index("c")
    base = wid * W_CNT
    pltpu.sync_copy(coords_hbm, tab_v)

    pltpu.sync_copy(nis_hbm.at[pl.ds(base, W_CNT)], idx_v)

    def body_i(t, _):
        for u in range(SC_UNROLL):
            off = (t * SC_UNROLL + u) * 16
            idx16 = idx_v[pl.ds(off, 16)]
            out_v[pl.ds(off, 16)] = plsc.load_gather(tab_v, [idx16])
        return 0

    lax.fori_loop(0, W_CNT // (16 * SC_UNROLL), body_i, 0)

    pltpu.sync_copy(njs_hbm.at[pl.ds(base, W_CNT)], idx_v)

    def body_j(t, _):
        for u in range(SC_UNROLL):
            off = (t * SC_UNROLL + u) * 16
            idx16 = idx_v[pl.ds(off, 16)]
            cj = plsc.load_gather(tab_v, [idx16])
            out_v[pl.ds(off, 16)] = out_v[pl.ds(off, 16)] | (cj << 8)
        return 0

    lax.fori_loop(0, W_CNT // (16 * SC_UNROLL), body_j, 0)
    pltpu.sync_copy(out_v, key_hbm.at[pl.ds(base, W_CNT)])


def _sc_gather(coords, nis_p, njs_p):
    call = functools.partial(
        pl.kernel,
        mesh=plsc.VectorSubcoreMesh(core_axis_name="c", subcore_axis_name="s"),
        compiler_params=pltpu.CompilerParams(needs_layout_passes=False),
        out_type=jax.ShapeDtypeStruct((EP,), jnp.int32),
        scratch_types=[
            pltpu.VMEM((N_NODES,), jnp.int32),
            pltpu.VMEM((W_CNT,), jnp.int32),
            pltpu.VMEM((W_CNT,), jnp.int32),
        ],
    )(_sc_body)
    return call(coords, nis_p, njs_p)


def _prep_body(embt_ref, wlt_ref, pct_ref):
    embt = embt_ref[...]
    pct_ref[:, 0:128] = jnp.dot(
        wlt_ref[:, 0:64], embt, preferred_element_type=jnp.float32,
        precision=lax.Precision.HIGHEST,
    )
    pct_ref[:, 128:256] = jnp.dot(
        wlt_ref[:, 64:128], embt, preferred_element_type=jnp.float32,
        precision=lax.Precision.HIGHEST,
    )


def _edge_body(key_ref, rbf_ref, pct_ref, wc_ref, b0_ref, wlin_ref,
               bl_ref, e1_ref, e2_ref):
    key = key_ref[0]
    ci16 = (key & 255).astype(jnp.int16)
    cj16 = (key >> 8).astype(jnp.int16) + jnp.int16(128)
    siota = lax.broadcasted_iota(jnp.int16, (256, 128), 0)
    pieces = []
    for r in range(KROWS):
        bci = jnp.broadcast_to(ci16[r:r + 1, :], (256, 128))
        bcj = jnp.broadcast_to(cj16[r:r + 1, :], (256, 128))
        sel = (siota == bci) | (siota == bcj)
        pieces.append(jnp.where(sel, jnp.bfloat16(1), jnp.bfloat16(0)))
    oht = jnp.concatenate(pieces, axis=1)
    g = lax.dot_general(oht, pct_ref[...].astype(jnp.bfloat16),
                        dimension_numbers=(((0,), (1,)), ((), ())),
                        preferred_element_type=jnp.float32)

    rbf = rbf_ref[...]
    h = jnp.dot(rbf, wc_ref[...], preferred_element_type=jnp.float32)
    h0 = h[:, 0:64] + b0_ref[...]
    rbf0 = h0 * (1.0 / (1.0 + jnp.exp(-h0)))

    h1 = g + jnp.dot(rbf0, wlin_ref[128:192, :],
                     preferred_element_type=jnp.float32) + bl_ref[...]
    e1 = h1 * (1.0 / (1.0 + jnp.exp(-h1)))
    e1_ref[...] = e1
    e2_ref[...] = h[:, 64:128] * e1


def _edge_call(key2, rbf, pct, wc, b0, wlin, bl):
    grid = (E // ET,)
    return pl.pallas_call(
        _edge_body,
        grid=grid,
        in_specs=[
            pl.BlockSpec((1, KROWS, 128), lambda i: (i, 0, 0)),
            pl.BlockSpec((ET, NR), lambda i: (i, 0)),
            pl.BlockSpec((H, 256), lambda i: (0, 0)),
            pl.BlockSpec((NR, 128), lambda i: (0, 0)),
            pl.BlockSpec((1, H), lambda i: (0, 0)),
            pl.BlockSpec((3 * H, H), lambda i: (0, 0)),
            pl.BlockSpec((1, H), lambda i: (0, 0)),
        ],
        out_specs=[
            pl.BlockSpec((ET, H), lambda i: (i, 0)),
            pl.BlockSpec((ET, H), lambda i: (i, 0)),
        ],
        out_shape=[
            jax.ShapeDtypeStruct((E, H), jnp.float32),
            jax.ShapeDtypeStruct((E, H), jnp.float32),
        ],
    )(key2, rbf, pct, wc, b0, wlin, bl)


def kernel(coords, rbf, node_is, node_js, emb_table, W_rbf0, b_rbf0, W_lin,
           b_lin, W_rbf1):
    nis_p = jnp.concatenate(
        [node_is, jnp.zeros((EP - E,), node_is.dtype)]).astype(jnp.int32)
    njs_p = jnp.concatenate(
        [node_js, jnp.zeros((EP - E,), node_js.dtype)]).astype(jnp.int32)
    key_p = _sc_gather(coords.astype(jnp.int32), nis_p, njs_p)
    key2 = key_p[:E].reshape(E // ET, KROWS, 128)

    embt = jnp.concatenate(
        [emb_table, jnp.zeros((128 - NT, H), emb_table.dtype)]).T
    pct = pl.pallas_call(
        _prep_body,
        out_shape=jax.ShapeDtypeStruct((H, 256), jnp.float32),
    )(embt, W_lin.T)

    wc = jnp.concatenate([W_rbf0, W_rbf1], axis=1)
    b0 = b_rbf0.reshape(1, H)
    bl = b_lin.reshape(1, H)
    return _edge_call(key2, rbf, pct, wc, b0, W_lin, bl)


# speedup vs baseline: 12.5879x; 4.4043x over previous
import functools

import jax
import jax.numpy as jnp
from jax import lax
from jax.experimental import pallas as pl
from jax.experimental.pallas import tpu as pltpu
from jax.experimental.pallas import tpu_sc as plsc

E = 800_000
N_NODES = 50_000
H = 64
NR = 16
NT = 95

NW = 32
W_CNT = 25_088
EP = NW * W_CNT
SC_UNROLL = 8

ET = 6_400
KROWS = ET // 128


def _sc_body(coords_hbm, nis_hbm, njs_hbm, key_hbm, tab_v, idx_v, out_v):
    wid = lax.axis_index("s") * 2 + lax.axis_index("c")
    base = wid * W_CNT
    pltpu.sync_copy(coords_hbm, tab_v)

    pltpu.sync_copy(nis_hbm.at[pl.ds(base, W_CNT)], idx_v)

    def body_i(t, _):
        for u in range(SC_UNROLL):
            off = (t * SC_UNROLL + u) * 16
            idx16 = idx_v[pl.ds(off, 16)]
            out_v[pl.ds(off, 16)] = plsc.load_gather(tab_v, [idx16])
        return 0

    lax.fori_loop(0, W_CNT // (16 * SC_UNROLL), body_i, 0)

    pltpu.sync_copy(njs_hbm.at[pl.ds(base, W_CNT)], idx_v)

    def body_j(t, _):
        for u in range(SC_UNROLL):
            off = (t * SC_UNROLL + u) * 16
            idx16 = idx_v[pl.ds(off, 16)]
            cj = plsc.load_gather(tab_v, [idx16])
            out_v[pl.ds(off, 16)] = out_v[pl.ds(off, 16)] | (cj << 8)
        return 0

    lax.fori_loop(0, W_CNT // (16 * SC_UNROLL), body_j, 0)
    pltpu.sync_copy(out_v, key_hbm.at[pl.ds(base, W_CNT)])


def _sc_gather(coords, nis_p, njs_p):
    call = functools.partial(
        pl.kernel,
        mesh=plsc.VectorSubcoreMesh(core_axis_name="c", subcore_axis_name="s"),
        compiler_params=pltpu.CompilerParams(needs_layout_passes=False),
        out_type=jax.ShapeDtypeStruct((EP,), jnp.int32),
        scratch_types=[
            pltpu.VMEM((N_NODES,), jnp.int32),
            pltpu.VMEM((W_CNT,), jnp.int32),
            pltpu.VMEM((W_CNT,), jnp.int32),
        ],
    )(_sc_body)
    return call(coords, nis_p, njs_p)


def _prep_body(embt_ref, wlt_ref, pct_ref):
    embt = embt_ref[...]
    pct_ref[:, 0:128] = jnp.dot(
        wlt_ref[:, 0:64], embt, preferred_element_type=jnp.float32,
        precision=lax.Precision.HIGHEST,
    )
    pct_ref[:, 128:256] = jnp.dot(
        wlt_ref[:, 64:128], embt, preferred_element_type=jnp.float32,
        precision=lax.Precision.HIGHEST,
    )


def _edge_body(key_ref, rbft_ref, pct_ref, wct_ref, b0_ref, wlt_ref,
               bl_ref, e1_ref, e2_ref):
    key = key_ref[0]
    ci16 = (key & 255).astype(jnp.int16)
    cj16 = (key >> 8).astype(jnp.int16) + jnp.int16(128)
    siota = lax.broadcasted_iota(jnp.int16, (256, 128), 0)
    pieces = []
    for r in range(KROWS):
        bci = jnp.broadcast_to(ci16[r:r + 1, :], (256, 128))
        bcj = jnp.broadcast_to(cj16[r:r + 1, :], (256, 128))
        sel = (siota == bci) | (siota == bcj)
        pieces.append(jnp.where(sel, jnp.bfloat16(1), jnp.bfloat16(0)))
    oht = jnp.concatenate(pieces, axis=1)
    gt = jnp.dot(pct_ref[...].astype(jnp.bfloat16), oht,
                 preferred_element_type=jnp.float32)

    rbft = rbft_ref[...]
    ht = jnp.dot(wct_ref[...], rbft,
                 preferred_element_type=jnp.float32)
    h0 = ht[0:64, :] + b0_ref[...]
    rbf0 = h0 * (1.0 / (1.0 + jnp.exp(-h0)))

    h1 = gt + jnp.dot(wlt_ref[:, 128:192], rbf0,
                      preferred_element_type=jnp.float32) + bl_ref[...]
    e1 = h1 * (1.0 / (1.0 + jnp.exp(-h1)))
    e1_ref[...] = e1
    e2_ref[...] = ht[64:128, :] * e1


def _edge_call(key2, rbft, pct, wct, b0, wlt, bl):
    grid = (E // ET,)
    return pl.pallas_call(
        _edge_body,
        grid=grid,
        in_specs=[
            pl.BlockSpec((1, KROWS, 128), lambda i: (i, 0, 0)),
            pl.BlockSpec((NR, ET), lambda i: (0, i)),
            pl.BlockSpec((H, 256), lambda i: (0, 0)),
            pl.BlockSpec((128, NR), lambda i: (0, 0)),
            pl.BlockSpec((H, 1), lambda i: (0, 0)),
            pl.BlockSpec((H, 3 * H), lambda i: (0, 0)),
            pl.BlockSpec((H, 1), lambda i: (0, 0)),
        ],
        out_specs=[
            pl.BlockSpec((H, ET), lambda i: (0, i)),
            pl.BlockSpec((H, ET), lambda i: (0, i)),
        ],
        out_shape=[
            jax.ShapeDtypeStruct((H, E), jnp.float32),
            jax.ShapeDtypeStruct((H, E), jnp.float32),
        ],
    )(key2, rbft, pct, wct, b0, wlt, bl)


def kernel(coords, rbf, node_is, node_js, emb_table, W_rbf0, b_rbf0, W_lin,
           b_lin, W_rbf1):
    nis_p = jnp.concatenate(
        [node_is, jnp.zeros((EP - E,), node_is.dtype)]).astype(jnp.int32)
    njs_p = jnp.concatenate(
        [node_js, jnp.zeros((EP - E,), node_js.dtype)]).astype(jnp.int32)
    key_p = _sc_gather(coords.astype(jnp.int32), nis_p, njs_p)
    key2 = key_p[:E].reshape(E // ET, KROWS, 128)

    embt = jnp.concatenate(
        [emb_table, jnp.zeros((128 - NT, H), emb_table.dtype)]).T
    pct = pl.pallas_call(
        _prep_body,
        out_shape=jax.ShapeDtypeStruct((H, 256), jnp.float32),
    )(embt, W_lin.T)

    wct = jnp.concatenate([W_rbf0.T, W_rbf1.T], axis=0)
    b0 = b_rbf0.reshape(H, 1)
    bl = b_lin.reshape(H, 1)
    e1t, e2t = _edge_call(key2, rbf.T, pct, wct, b0, W_lin.T, bl)
    return e1t.T, e2t.T


# speedup vs baseline: 13.0462x; 1.0364x over previous
import functools

import jax
import jax.numpy as jnp
from jax import lax
from jax.experimental import pallas as pl
from jax.experimental.pallas import tpu as pltpu
from jax.experimental.pallas import tpu_sc as plsc

E = 800_000
N_NODES = 50_000
H = 64
NR = 16
NT = 95

NW = 32
W_CNT = 25_088
EP = NW * W_CNT
SC_UNROLL = 8

ET = 16_000
KROWS = ET // 128


def _sc_body(coords_hbm, nis_hbm, njs_hbm, key_hbm, tab_v, idx_v, out_v):
    wid = lax.axis_index("s") * 2 + lax.axis_index("c")
    base = wid * W_CNT
    pltpu.sync_copy(coords_hbm, tab_v)

    pltpu.sync_copy(nis_hbm.at[pl.ds(base, W_CNT)], idx_v)

    def body_i(t, _):
        for u in range(SC_UNROLL):
            off = (t * SC_UNROLL + u) * 16
            idx16 = idx_v[pl.ds(off, 16)]
            out_v[pl.ds(off, 16)] = plsc.load_gather(tab_v, [idx16])
        return 0

    lax.fori_loop(0, W_CNT // (16 * SC_UNROLL), body_i, 0)

    pltpu.sync_copy(njs_hbm.at[pl.ds(base, W_CNT)], idx_v)

    def body_j(t, _):
        for u in range(SC_UNROLL):
            off = (t * SC_UNROLL + u) * 16
            idx16 = idx_v[pl.ds(off, 16)]
            cj = plsc.load_gather(tab_v, [idx16])
            out_v[pl.ds(off, 16)] = out_v[pl.ds(off, 16)] | (cj << 8)
        return 0

    lax.fori_loop(0, W_CNT // (16 * SC_UNROLL), body_j, 0)
    pltpu.sync_copy(out_v, key_hbm.at[pl.ds(base, W_CNT)])


def _sc_gather(coords, nis_p, njs_p):
    call = functools.partial(
        pl.kernel,
        mesh=plsc.VectorSubcoreMesh(core_axis_name="c", subcore_axis_name="s"),
        compiler_params=pltpu.CompilerParams(needs_layout_passes=False),
        out_type=jax.ShapeDtypeStruct((EP,), jnp.int32),
        scratch_types=[
            pltpu.VMEM((N_NODES,), jnp.int32),
            pltpu.VMEM((W_CNT,), jnp.int32),
            pltpu.VMEM((W_CNT,), jnp.int32),
        ],
    )(_sc_body)
    return call(coords, nis_p, njs_p)


def _prep_body(embt_ref, wlt_ref, pct_ref):
    embt = embt_ref[...]
    pct_ref[:, 0:128] = jnp.dot(
        wlt_ref[:, 0:64], embt, preferred_element_type=jnp.float32,
        precision=lax.Precision.HIGHEST,
    )
    pct_ref[:, 128:256] = jnp.dot(
        wlt_ref[:, 64:128], embt, preferred_element_type=jnp.float32,
        precision=lax.Precision.HIGHEST,
    )


def _edge_body(key_ref, rbft_ref, pct_ref, wct_ref, b0_ref, wlt_ref,
               bl_ref, e1_ref, e2_ref):
    key = key_ref[0]
    ci16 = (key & 255).astype(jnp.int16)
    cj16 = (key >> 8).astype(jnp.int16) + jnp.int16(128)
    siota = lax.broadcasted_iota(jnp.int16, (256, 128), 0)
    pieces = []
    for r in range(KROWS):
        bci = jnp.broadcast_to(ci16[r:r + 1, :], (256, 128))
        bcj = jnp.broadcast_to(cj16[r:r + 1, :], (256, 128))
        sel = (siota == bci) | (siota == bcj)
        pieces.append(jnp.where(sel, jnp.bfloat16(1), jnp.bfloat16(0)))
    oht = jnp.concatenate(pieces, axis=1)
    gt = jnp.dot(pct_ref[...].astype(jnp.bfloat16), oht,
                 preferred_element_type=jnp.float32)

    rbft = rbft_ref[...]
    ht = jnp.dot(wct_ref[...], rbft,
                 preferred_element_type=jnp.float32)
    h0 = ht[0:64, :] + b0_ref[...]
    rbf0 = h0 * (1.0 / (1.0 + jnp.exp(-h0)))

    h1 = gt + jnp.dot(wlt_ref[:, 128:192], rbf0,
                      preferred_element_type=jnp.float32) + bl_ref[...]
    e1 = h1 * (1.0 / (1.0 + jnp.exp(-h1)))
    e1_ref[...] = e1
    e2_ref[...] = ht[64:128, :] * e1


def _edge_call(key2, rbft, pct, wct, b0, wlt, bl):
    grid = (E // ET,)
    return pl.pallas_call(
        _edge_body,
        grid=grid,
        in_specs=[
            pl.BlockSpec((1, KROWS, 128), lambda i: (i, 0, 0)),
            pl.BlockSpec((NR, ET), lambda i: (0, i)),
            pl.BlockSpec((H, 256), lambda i: (0, 0)),
            pl.BlockSpec((128, NR), lambda i: (0, 0)),
            pl.BlockSpec((H, 1), lambda i: (0, 0)),
            pl.BlockSpec((H, 3 * H), lambda i: (0, 0)),
            pl.BlockSpec((H, 1), lambda i: (0, 0)),
        ],
        out_specs=[
            pl.BlockSpec((H, ET), lambda i: (0, i)),
            pl.BlockSpec((H, ET), lambda i: (0, i)),
        ],
        out_shape=[
            jax.ShapeDtypeStruct((H, E), jnp.float32),
            jax.ShapeDtypeStruct((H, E), jnp.float32),
        ],
    )(key2, rbft, pct, wct, b0, wlt, bl)


def kernel(coords, rbf, node_is, node_js, emb_table, W_rbf0, b_rbf0, W_lin,
           b_lin, W_rbf1):
    nis_p = jnp.concatenate(
        [node_is, jnp.zeros((EP - E,), node_is.dtype)]).astype(jnp.int32)
    njs_p = jnp.concatenate(
        [node_js, jnp.zeros((EP - E,), node_js.dtype)]).astype(jnp.int32)
    key_p = _sc_gather(coords.astype(jnp.int32), nis_p, njs_p)
    key2 = key_p[:E].reshape(E // ET, KROWS, 128)

    embt = jnp.concatenate(
        [emb_table, jnp.zeros((128 - NT, H), emb_table.dtype)]).T
    pct = pl.pallas_call(
        _prep_body,
        out_shape=jax.ShapeDtypeStruct((H, 256), jnp.float32),
    )(embt, W_lin.T)

    wct = jnp.concatenate([W_rbf0.T, W_rbf1.T], axis=0)
    b0 = b_rbf0.reshape(H, 1)
    bl = b_lin.reshape(H, 1)
    e1t, e2t = _edge_call(key2, rbf.T, pct, wct, b0, W_lin.T, bl)
    return e1t.T, e2t.T


# speedup vs baseline: 13.6275x; 1.0446x over previous
import functools

import jax
import jax.numpy as jnp
from jax import lax
from jax.experimental import pallas as pl
from jax.experimental.pallas import tpu as pltpu
from jax.experimental.pallas import tpu_sc as plsc

E = 800_000
N_NODES = 50_000
H = 64
NR = 16
NT = 95

NW = 32
W_CNT = 25_088
EP = NW * W_CNT
SC_UNROLL = 8

ET = 16_000
KROWS = ET // 128


def _sc_body(coords_hbm, nis_hbm, njs_hbm, key_hbm, tab_v, idx_v, out_v):
    wid = lax.axis_index("s") * 2 + lax.axis_index("c")
    base = wid * W_CNT
    pltpu.sync_copy(coords_hbm, tab_v)

    pltpu.sync_copy(nis_hbm.at[pl.ds(base, W_CNT)], idx_v)

    @plsc.parallel_loop(0, W_CNT // 16, unroll=SC_UNROLL)
    def body_i(t):
        off = t * 16
        idx16 = idx_v[pl.ds(off, 16)]
        out_v[pl.ds(off, 16)] = plsc.load_gather(tab_v, [idx16])

    pltpu.sync_copy(njs_hbm.at[pl.ds(base, W_CNT)], idx_v)

    @plsc.parallel_loop(0, W_CNT // 16, unroll=SC_UNROLL)
    def body_j(t):
        off = t * 16
        idx16 = idx_v[pl.ds(off, 16)]
        cj = plsc.load_gather(tab_v, [idx16])
        out_v[pl.ds(off, 16)] = out_v[pl.ds(off, 16)] | (cj << 8)

    pltpu.sync_copy(out_v, key_hbm.at[pl.ds(base, W_CNT)])


def _sc_gather(coords, nis_p, njs_p):
    call = functools.partial(
        pl.kernel,
        mesh=plsc.VectorSubcoreMesh(core_axis_name="c", subcore_axis_name="s"),
        compiler_params=pltpu.CompilerParams(needs_layout_passes=False),
        out_type=jax.ShapeDtypeStruct((EP,), jnp.int32),
        scratch_types=[
            pltpu.VMEM((N_NODES,), jnp.int32),
            pltpu.VMEM((W_CNT,), jnp.int32),
            pltpu.VMEM((W_CNT,), jnp.int32),
        ],
    )(_sc_body)
    return call(coords, nis_p, njs_p)


def _prep_body(embt_ref, wlt_ref, pct_ref):
    embt = embt_ref[...]
    pct_ref[:, 0:128] = jnp.dot(
        wlt_ref[:, 0:64], embt, preferred_element_type=jnp.float32,
        precision=lax.Precision.HIGHEST,
    )
    pct_ref[:, 128:256] = jnp.dot(
        wlt_ref[:, 64:128], embt, preferred_element_type=jnp.float32,
        precision=lax.Precision.HIGHEST,
    )


def _edge_body(key_ref, rbft_ref, pct_ref, wct_ref, b0_ref, wlt_ref,
               bl_ref, e1_ref, e2_ref):
    key = key_ref[0]
    ci16 = (key & 255).astype(jnp.int16)
    cj16 = (key >> 8).astype(jnp.int16) + jnp.int16(128)
    siota = lax.broadcasted_iota(jnp.int16, (256, 128), 0)
    pieces = []
    for r in range(KROWS):
        bci = jnp.broadcast_to(ci16[r:r + 1, :], (256, 128))
        bcj = jnp.broadcast_to(cj16[r:r + 1, :], (256, 128))
        sel = (siota == bci) | (siota == bcj)
        pieces.append(jnp.where(sel, jnp.bfloat16(1), jnp.bfloat16(0)))
    oht = jnp.concatenate(pieces, axis=1)
    gt = jnp.dot(pct_ref[...].astype(jnp.bfloat16), oht,
                 preferred_element_type=jnp.float32)

    rbft = rbft_ref[...]
    ht = jnp.dot(wct_ref[...], rbft,
                 preferred_element_type=jnp.float32)
    h0 = ht[0:64, :] + b0_ref[...]
    rbf0 = h0 * (1.0 / (1.0 + jnp.exp(-h0)))

    h1 = gt + jnp.dot(wlt_ref[:, 128:192], rbf0,
                      preferred_element_type=jnp.float32) + bl_ref[...]
    e1 = h1 * (1.0 / (1.0 + jnp.exp(-h1)))
    e1_ref[...] = e1
    e2_ref[...] = ht[64:128, :] * e1


def _edge_call(key2, rbft, pct, wct, b0, wlt, bl):
    grid = (E // ET,)
    return pl.pallas_call(
        _edge_body,
        grid=grid,
        in_specs=[
            pl.BlockSpec((1, KROWS, 128), lambda i: (i, 0, 0)),
            pl.BlockSpec((NR, ET), lambda i: (0, i)),
            pl.BlockSpec((H, 256), lambda i: (0, 0)),
            pl.BlockSpec((128, NR), lambda i: (0, 0)),
            pl.BlockSpec((H, 1), lambda i: (0, 0)),
            pl.BlockSpec((H, 3 * H), lambda i: (0, 0)),
            pl.BlockSpec((H, 1), lambda i: (0, 0)),
        ],
        out_specs=[
            pl.BlockSpec((H, ET), lambda i: (0, i)),
            pl.BlockSpec((H, ET), lambda i: (0, i)),
        ],
        out_shape=[
            jax.ShapeDtypeStruct((H, E), jnp.float32),
            jax.ShapeDtypeStruct((H, E), jnp.float32),
        ],
    )(key2, rbft, pct, wct, b0, wlt, bl)


def kernel(coords, rbf, node_is, node_js, emb_table, W_rbf0, b_rbf0, W_lin,
           b_lin, W_rbf1):
    nis_p = jnp.concatenate(
        [node_is, jnp.zeros((EP - E,), node_is.dtype)]).astype(jnp.int32)
    njs_p = jnp.concatenate(
        [node_js, jnp.zeros((EP - E,), node_js.dtype)]).astype(jnp.int32)
    key_p = _sc_gather(coords.astype(jnp.int32), nis_p, njs_p)
    key2 = key_p[:E].reshape(E // ET, KROWS, 128)

    embt = jnp.concatenate(
        [emb_table, jnp.zeros((128 - NT, H), emb_table.dtype)]).T
    pct = pl.pallas_call(
        _prep_body,
        out_shape=jax.ShapeDtypeStruct((H, 256), jnp.float32),
    )(embt, W_lin.T)

    wct = jnp.concatenate([W_rbf0.T, W_rbf1.T], axis=0)
    b0 = b_rbf0.reshape(H, 1)
    bl = b_lin.reshape(H, 1)
    e1t, e2t = _edge_call(key2, rbf.T, pct, wct, b0, W_lin.T, bl)
    return e1t.T, e2t.T


# speedup vs baseline: 13.8525x; 1.0165x over previous
import functools

import jax
import jax.numpy as jnp
from jax import lax
from jax.experimental import pallas as pl
from jax.experimental.pallas import tpu as pltpu
from jax.experimental.pallas import tpu_sc as plsc

E = 800_000
N_NODES = 50_000
H = 64
NR = 16
NT = 95

NW = 32
W_CNT = 25_088
EP = NW * W_CNT
SC_UNROLL = 8

ET = 32_000
KROWS = ET // 128


def _sc_body(coords_hbm, nis_hbm, njs_hbm, key_hbm, tab_v, idx_v, out_v):
    wid = lax.axis_index("s") * 2 + lax.axis_index("c")
    base = wid * W_CNT
    pltpu.sync_copy(coords_hbm, tab_v)

    pltpu.sync_copy(nis_hbm.at[pl.ds(base, W_CNT)], idx_v)

    @plsc.parallel_loop(0, W_CNT // 16, unroll=SC_UNROLL)
    def body_i(t):
        off = t * 16
        idx16 = idx_v[pl.ds(off, 16)]
        out_v[pl.ds(off, 16)] = plsc.load_gather(tab_v, [idx16])

    pltpu.sync_copy(njs_hbm.at[pl.ds(base, W_CNT)], idx_v)

    @plsc.parallel_loop(0, W_CNT // 16, unroll=SC_UNROLL)
    def body_j(t):
        off = t * 16
        idx16 = idx_v[pl.ds(off, 16)]
        cj = plsc.load_gather(tab_v, [idx16])
        out_v[pl.ds(off, 16)] = out_v[pl.ds(off, 16)] | (cj << 8)

    pltpu.sync_copy(out_v, key_hbm.at[pl.ds(base, W_CNT)])


def _sc_gather(coords, nis_p, njs_p):
    call = functools.partial(
        pl.kernel,
        mesh=plsc.VectorSubcoreMesh(core_axis_name="c", subcore_axis_name="s"),
        compiler_params=pltpu.CompilerParams(needs_layout_passes=False),
        out_type=jax.ShapeDtypeStruct((EP,), jnp.int32),
        scratch_types=[
            pltpu.VMEM((N_NODES,), jnp.int32),
            pltpu.VMEM((W_CNT,), jnp.int32),
            pltpu.VMEM((W_CNT,), jnp.int32),
        ],
    )(_sc_body)
    return call(coords, nis_p, njs_p)


def _prep_body(embt_ref, wlt_ref, pct_ref):
    embt = embt_ref[...]
    pct_ref[:, 0:128] = jnp.dot(
        wlt_ref[:, 0:64], embt, preferred_element_type=jnp.float32,
        precision=lax.Precision.HIGHEST,
    )
    pct_ref[:, 128:256] = jnp.dot(
        wlt_ref[:, 64:128], embt, preferred_element_type=jnp.float32,
        precision=lax.Precision.HIGHEST,
    )


def _edge_body(key_ref, rbft_ref, pct_ref, wct_ref, b0_ref, wlt_ref,
               bl_ref, e1_ref, e2_ref):
    key = key_ref[0]
    ci16 = (key & 255).astype(jnp.int16)
    cj16 = (key >> 8).astype(jnp.int16) + jnp.int16(128)
    siota = lax.broadcasted_iota(jnp.int16, (256, 128), 0)
    pieces = []
    for r in range(KROWS):
        bci = jnp.broadcast_to(ci16[r:r + 1, :], (256, 128))
        bcj = jnp.broadcast_to(cj16[r:r + 1, :], (256, 128))
        sel = (siota == bci) | (siota == bcj)
        pieces.append(jnp.where(sel, jnp.bfloat16(1), jnp.bfloat16(0)))
    oht = jnp.concatenate(pieces, axis=1)
    gt = jnp.dot(pct_ref[...].astype(jnp.bfloat16), oht,
                 preferred_element_type=jnp.float32)

    rbft = rbft_ref[...]
    ht = jnp.dot(wct_ref[...], rbft,
                 preferred_element_type=jnp.float32)
    h0 = ht[0:64, :] + b0_ref[...]
    rbf0 = h0 * (1.0 / (1.0 + jnp.exp(-h0)))

    h1 = gt + jnp.dot(wlt_ref[:, 128:192], rbf0,
                      preferred_element_type=jnp.float32) + bl_ref[...]
    e1 = h1 * (1.0 / (1.0 + jnp.exp(-h1)))
    e1_ref[...] = e1
    e2_ref[...] = ht[64:128, :] * e1


def _edge_call(key2, rbft, pct, wct, b0, wlt, bl):
    grid = (E // ET,)
    return pl.pallas_call(
        _edge_body,
        grid=grid,
        in_specs=[
            pl.BlockSpec((1, KROWS, 128), lambda i: (i, 0, 0)),
            pl.BlockSpec((NR, ET), lambda i: (0, i)),
            pl.BlockSpec((H, 256), lambda i: (0, 0)),
            pl.BlockSpec((128, NR), lambda i: (0, 0)),
            pl.BlockSpec((H, 1), lambda i: (0, 0)),
            pl.BlockSpec((H, 3 * H), lambda i: (0, 0)),
            pl.BlockSpec((H, 1), lambda i: (0, 0)),
        ],
        out_specs=[
            pl.BlockSpec((H, ET), lambda i: (0, i)),
            pl.BlockSpec((H, ET), lambda i: (0, i)),
        ],
        out_shape=[
            jax.ShapeDtypeStruct((H, E), jnp.float32),
            jax.ShapeDtypeStruct((H, E), jnp.float32),
        ],
    )(key2, rbft, pct, wct, b0, wlt, bl)


def kernel(coords, rbf, node_is, node_js, emb_table, W_rbf0, b_rbf0, W_lin,
           b_lin, W_rbf1):
    nis_p = jnp.concatenate(
        [node_is, jnp.zeros((EP - E,), node_is.dtype)]).astype(jnp.int32)
    njs_p = jnp.concatenate(
        [node_js, jnp.zeros((EP - E,), node_js.dtype)]).astype(jnp.int32)
    key_p = _sc_gather(coords.astype(jnp.int32), nis_p, njs_p)
    key2 = key_p[:E].reshape(E // ET, KROWS, 128)

    embt = jnp.concatenate(
        [emb_table, jnp.zeros((128 - NT, H), emb_table.dtype)]).T
    pct = pl.pallas_call(
        _prep_body,
        out_shape=jax.ShapeDtypeStruct((H, 256), jnp.float32),
    )(embt, W_lin.T)

    wct = jnp.concatenate([W_rbf0.T, W_rbf1.T], axis=0)
    b0 = b_rbf0.reshape(H, 1)
    bl = b_lin.reshape(H, 1)
    e1t, e2t = _edge_call(key2, rbf.T, pct, wct, b0, W_lin.T, bl)
    return e1t.T, e2t.T


# speedup vs baseline: 14.4969x; 1.0465x over previous
import functools

import jax
import jax.numpy as jnp
from jax import lax
from jax.experimental import pallas as pl
from jax.experimental.pallas import tpu as pltpu
from jax.experimental.pallas import tpu_sc as plsc

E = 800_000
N_NODES = 50_000
H = 64
NR = 16
NT = 95

NW = 32
W_CNT = 25_088
EP = NW * W_CNT
SC_UNROLL = 8

ET = 32_000
KROWS = ET // 128


def _sc_body(coords_hbm, nis_hbm, njs_hbm, key_hbm, tab_v, idx_v, out_v):
    wid = lax.axis_index("s") * 2 + lax.axis_index("c")
    base = wid * W_CNT
    pltpu.sync_copy(coords_hbm, tab_v)

    pltpu.sync_copy(nis_hbm.at[pl.ds(base, W_CNT)], idx_v)

    @plsc.parallel_loop(0, W_CNT // 16, unroll=SC_UNROLL)
    def body_i(t):
        off = t * 16
        idx16 = idx_v[pl.ds(off, 16)]
        out_v[pl.ds(off, 16)] = plsc.load_gather(tab_v, [idx16])

    pltpu.sync_copy(njs_hbm.at[pl.ds(base, W_CNT)], idx_v)

    @plsc.parallel_loop(0, W_CNT // 16, unroll=SC_UNROLL)
    def body_j(t):
        off = t * 16
        idx16 = idx_v[pl.ds(off, 16)]
        cj = plsc.load_gather(tab_v, [idx16])
        out_v[pl.ds(off, 16)] = out_v[pl.ds(off, 16)] | (cj << 8)

    pltpu.sync_copy(out_v, key_hbm.at[pl.ds(base, W_CNT)])


def _sc_gather(coords, nis_p, njs_p):
    call = functools.partial(
        pl.kernel,
        mesh=plsc.VectorSubcoreMesh(core_axis_name="c", subcore_axis_name="s"),
        compiler_params=pltpu.CompilerParams(needs_layout_passes=False),
        out_type=jax.ShapeDtypeStruct((EP,), jnp.int32),
        scratch_types=[
            pltpu.VMEM((N_NODES,), jnp.int32),
            pltpu.VMEM((W_CNT,), jnp.int32),
            pltpu.VMEM((W_CNT,), jnp.int32),
        ],
    )(_sc_body)
    return call(coords, nis_p, njs_p)


def _prep_body(embt_ref, wlt_ref, pct_ref):
    embt = embt_ref[...]
    pct_ref[:, 0:128] = jnp.dot(
        wlt_ref[:, 0:64], embt, preferred_element_type=jnp.float32,
        precision=lax.Precision.HIGHEST,
    )
    pct_ref[:, 128:256] = jnp.dot(
        wlt_ref[:, 64:128], embt, preferred_element_type=jnp.float32,
        precision=lax.Precision.HIGHEST,
    )


def _edge_body(key_ref, rbft_ref, pct_ref, wct_ref, b0_ref, wlt_ref,
               bl_ref, e1_ref, e2_ref):
    key = key_ref[0]
    ci16 = (key & 255).astype(jnp.int16)
    cj16 = (key >> 8).astype(jnp.int16) + jnp.int16(128)
    siota = lax.broadcasted_iota(jnp.int16, (256, 128), 0)
    pieces = []
    for r in range(KROWS):
        bci = jnp.broadcast_to(ci16[r:r + 1, :], (256, 128))
        bcj = jnp.broadcast_to(cj16[r:r + 1, :], (256, 128))
        sel = (siota == bci) | (siota == bcj)
        pieces.append(jnp.where(sel, jnp.bfloat16(1), jnp.bfloat16(0)))
    oht = jnp.concatenate(pieces, axis=1)
    gt = jnp.dot(pct_ref[...].astype(jnp.bfloat16), oht,
                 preferred_element_type=jnp.float32)

    rbft = rbft_ref[...]
    ht = jnp.dot(wct_ref[...], rbft,
                 preferred_element_type=jnp.float32)
    t0 = (ht[0:64, :] + b0_ref[...]) * 0.5
    rbf0 = t0 * jnp.tanh(t0) + t0

    h1 = gt + jnp.dot(wlt_ref[:, 128:192], rbf0,
                      preferred_element_type=jnp.float32) + bl_ref[...]
    t1 = h1 * 0.5
    e1 = t1 * jnp.tanh(t1) + t1
    e1_ref[...] = e1
    e2_ref[...] = ht[64:128, :] * e1


def _edge_call(key2, rbft, pct, wct, b0, wlt, bl):
    grid = (E // ET,)
    return pl.pallas_call(
        _edge_body,
        grid=grid,
        in_specs=[
            pl.BlockSpec((1, KROWS, 128), lambda i: (i, 0, 0)),
            pl.BlockSpec((NR, ET), lambda i: (0, i)),
            pl.BlockSpec((H, 256), lambda i: (0, 0)),
            pl.BlockSpec((128, NR), lambda i: (0, 0)),
            pl.BlockSpec((H, 1), lambda i: (0, 0)),
            pl.BlockSpec((H, 3 * H), lambda i: (0, 0)),
            pl.BlockSpec((H, 1), lambda i: (0, 0)),
        ],
        out_specs=[
            pl.BlockSpec((H, ET), lambda i: (0, i)),
            pl.BlockSpec((H, ET), lambda i: (0, i)),
        ],
        out_shape=[
            jax.ShapeDtypeStruct((H, E), jnp.float32),
            jax.ShapeDtypeStruct((H, E), jnp.float32),
        ],
    )(key2, rbft, pct, wct, b0, wlt, bl)


def kernel(coords, rbf, node_is, node_js, emb_table, W_rbf0, b_rbf0, W_lin,
           b_lin, W_rbf1):
    nis_p = jnp.concatenate(
        [node_is, jnp.zeros((EP - E,), node_is.dtype)]).astype(jnp.int32)
    njs_p = jnp.concatenate(
        [node_js, jnp.zeros((EP - E,), node_js.dtype)]).astype(jnp.int32)
    key_p = _sc_gather(coords.astype(jnp.int32), nis_p, njs_p)
    key2 = key_p[:E].reshape(E // ET, KROWS, 128)

    embt = jnp.concatenate(
        [emb_table, jnp.zeros((128 - NT, H), emb_table.dtype)]).T
    pct = pl.pallas_call(
        _prep_body,
        out_shape=jax.ShapeDtypeStruct((H, 256), jnp.float32),
    )(embt, W_lin.T)

    wct = jnp.concatenate([W_rbf0.T, W_rbf1.T], axis=0)
    b0 = b_rbf0.reshape(H, 1)
    bl = b_lin.reshape(H, 1)
    e1t, e2t = _edge_call(key2, rbf.T, pct, wct, b0, W_lin.T, bl)
    return e1t.T, e2t.T


# speedup vs baseline: 14.7672x; 1.0186x over previous
import functools

import jax
import jax.numpy as jnp
from jax import lax
from jax.experimental import pallas as pl
from jax.experimental.pallas import tpu as pltpu
from jax.experimental.pallas import tpu_sc as plsc

E = 800_000
N_NODES = 50_000
H = 64
NR = 16
NT = 95

NW = 32
W_CNT = 25_088
W_STRIDE = E // NW
SC_UNROLL = 8

ET = 32_000
KROWS = ET // 128


def _sc_body(coords_hbm, nis_hbm, njs_hbm, key_hbm, tab_v, idx_v, out_v):
    wid = lax.axis_index("s") * 2 + lax.axis_index("c")
    base = jnp.minimum(wid * W_STRIDE, E - W_CNT)
    pltpu.sync_copy(coords_hbm, tab_v)

    pltpu.sync_copy(nis_hbm.at[pl.ds(base, W_CNT)], idx_v)

    @plsc.parallel_loop(0, W_CNT // 16, unroll=SC_UNROLL)
    def body_i(t):
        off = t * 16
        idx16 = idx_v[pl.ds(off, 16)]
        out_v[pl.ds(off, 16)] = plsc.load_gather(tab_v, [idx16])

    pltpu.sync_copy(njs_hbm.at[pl.ds(base, W_CNT)], idx_v)

    @plsc.parallel_loop(0, W_CNT // 16, unroll=SC_UNROLL)
    def body_j(t):
        off = t * 16
        idx16 = idx_v[pl.ds(off, 16)]
        cj = plsc.load_gather(tab_v, [idx16])
        out_v[pl.ds(off, 16)] = out_v[pl.ds(off, 16)] | (cj << 8)

    pltpu.sync_copy(out_v, key_hbm.at[pl.ds(base, W_CNT)])


def _sc_gather(coords, nis_p, njs_p):
    call = functools.partial(
        pl.kernel,
        mesh=plsc.VectorSubcoreMesh(core_axis_name="c", subcore_axis_name="s"),
        compiler_params=pltpu.CompilerParams(needs_layout_passes=False),
        out_type=jax.ShapeDtypeStruct((E,), jnp.int32),
        scratch_types=[
            pltpu.VMEM((N_NODES,), jnp.int32),
            pltpu.VMEM((W_CNT,), jnp.int32),
            pltpu.VMEM((W_CNT,), jnp.int32),
        ],
    )(_sc_body)
    return call(coords, nis_p, njs_p)


def _prep_body(embt_ref, wlt_ref, pct_ref):
    embt = embt_ref[...]
    pct_ref[:, 0:128] = jnp.dot(
        wlt_ref[:, 0:64], embt, preferred_element_type=jnp.float32,
        precision=lax.Precision.HIGHEST,
    )
    pct_ref[:, 128:256] = jnp.dot(
        wlt_ref[:, 64:128], embt, preferred_element_type=jnp.float32,
        precision=lax.Precision.HIGHEST,
    )


def _edge_body(key_ref, rbft_ref, pct_ref, wct_ref, b0_ref, wlt_ref,
               bl_ref, e1_ref, e2_ref):
    key = key_ref[0]
    ci16 = (key & 255).astype(jnp.int16)
    cj16 = (key >> 8).astype(jnp.int16) + jnp.int16(128)
    siota = lax.broadcasted_iota(jnp.int16, (256, 128), 0)
    pieces = []
    for r in range(KROWS):
        bci = jnp.broadcast_to(ci16[r:r + 1, :], (256, 128))
        bcj = jnp.broadcast_to(cj16[r:r + 1, :], (256, 128))
        sel = (siota == bci) | (siota == bcj)
        pieces.append(jnp.where(sel, jnp.bfloat16(1), jnp.bfloat16(0)))
    oht = jnp.concatenate(pieces, axis=1)
    gt = jnp.dot(pct_ref[...].astype(jnp.bfloat16), oht,
                 preferred_element_type=jnp.float32)

    rbft = rbft_ref[...]
    ht = jnp.dot(wct_ref[...], rbft,
                 preferred_element_type=jnp.float32)
    t0 = (ht[0:64, :] + b0_ref[...]) * 0.5
    rbf0 = t0 * jnp.tanh(t0) + t0

    h1 = gt + jnp.dot(wlt_ref[:, 128:192], rbf0,
                      preferred_element_type=jnp.float32) + bl_ref[...]
    t1 = h1 * 0.5
    e1 = t1 * jnp.tanh(t1) + t1
    e1_ref[...] = e1
    e2_ref[...] = ht[64:128, :] * e1


def _edge_call(key2, rbft, pct, wct, b0, wlt, bl):
    grid = (E // ET,)
    return pl.pallas_call(
        _edge_body,
        grid=grid,
        in_specs=[
            pl.BlockSpec((1, KROWS, 128), lambda i: (i, 0, 0)),
            pl.BlockSpec((NR, ET), lambda i: (0, i)),
            pl.BlockSpec((H, 256), lambda i: (0, 0)),
            pl.BlockSpec((128, NR), lambda i: (0, 0)),
            pl.BlockSpec((H, 1), lambda i: (0, 0)),
            pl.BlockSpec((H, 3 * H), lambda i: (0, 0)),
            pl.BlockSpec((H, 1), lambda i: (0, 0)),
        ],
        out_specs=[
            pl.BlockSpec((H, ET), lambda i: (0, i)),
            pl.BlockSpec((H, ET), lambda i: (0, i)),
        ],
        out_shape=[
            jax.ShapeDtypeStruct((H, E), jnp.float32),
            jax.ShapeDtypeStruct((H, E), jnp.float32),
        ],
    )(key2, rbft, pct, wct, b0, wlt, bl)


def kernel(coords, rbf, node_is, node_js, emb_table, W_rbf0, b_rbf0, W_lin,
           b_lin, W_rbf1):
    key_p = _sc_gather(coords.astype(jnp.int32), node_is.astype(jnp.int32),
                       node_js.astype(jnp.int32))
    key2 = key_p.reshape(E // ET, KROWS, 128)

    embt = jnp.concatenate(
        [emb_table, jnp.zeros((128 - NT, H), emb_table.dtype)]).T
    pct = pl.pallas_call(
        _prep_body,
        out_shape=jax.ShapeDtypeStruct((H, 256), jnp.float32),
    )(embt, W_lin.T)

    wct = jnp.concatenate([W_rbf0.T, W_rbf1.T], axis=0)
    b0 = b_rbf0.reshape(H, 1)
    bl = b_lin.reshape(H, 1)
    e1t, e2t = _edge_call(key2, rbf.T, pct, wct, b0, W_lin.T, bl)
    return e1t.T, e2t.T
